# Initial kernel scaffold; baseline (speedup 1.0000x reference)
import jax
import jax.numpy as jnp
from jax.experimental import pallas as pl


def kernel(flat_pair_enc, flat_pair_encA, group_index, phi1_W0, phi1_b0, phi1_W1, phi1_b1, phi1_W2, phi1_b2, g_W0, g_b0, g_W1, g_b1, g_W2, g_b2, phi2_W0, phi2_b0, phi2_W1, phi2_b1):
    raise NotImplementedError("write your pallas kernel here")


# speedup vs baseline: 2.2266x; 2.2266x over previous
import functools

import jax
import jax.numpy as jnp
from jax import lax
from jax.experimental import pallas as pl
from jax.experimental.pallas import tpu as pltpu
from jax.experimental.pallas import tpu_sc as plsc

G = 10000
GA = 10624
W = 128
BA = 512
RB = 128
BE = 1280


def _elu(x):
    return jnp.where(x > 0, x, jnp.exp(jnp.minimum(x, 0.0)) - 1.0)


def _phase_abc_body(enca_ref, gi_ref, w0_ref, b0_ref, w1_ref, b1_ref,
                    w2_ref, b2_ref, gw0_ref, gb0_ref, gw1_ref, gb1_ref,
                    gw2_ref, gb2_ref, key1_ref, qjt_ref, means_ref,
                    acc, cnt):
    f32 = jnp.float32
    pid = pl.program_id(0)
    nb = pl.num_programs(0)

    @pl.when(pid == 0)
    def _():
        acc[...] = jnp.zeros(acc.shape, f32)
        cnt[...] = jnp.zeros(cnt.shape, f32)

    x = enca_ref[...]
    h = _elu(jnp.dot(x, w0_ref[...], preferred_element_type=f32)
             + b0_ref[...])
    h = _elu(jnp.dot(h, w1_ref[...], preferred_element_type=f32)
             + b1_ref[...])
    k1 = jnp.dot(h, w2_ref[...], preferred_element_type=f32) + b2_ref[...]
    key1_ref[...] = k1

    gi_row = gi_ref[0]
    gmin = jnp.min(gi_row)
    gmax = jnp.max(gi_row)
    onesb = jnp.ones((BA, 8), f32)

    def cond(b_):
        return b_ <= gmax

    def body(b_):
        b8 = pl.multiple_of(b_, 8)
        ids = b8 + lax.broadcasted_iota(jnp.int32, (W, 1), 0)
        oh = (ids == gi_row).astype(f32)
        part = jnp.dot(oh, k1, preferred_element_type=f32)
        pc = jnp.dot(oh, onesb, preferred_element_type=f32)
        acc[pl.ds(b8, W), :] = acc[pl.ds(b8, W), :] + part
        cnt[pl.ds(b8, W), :] = cnt[pl.ds(b8, W), :] + pc
        return b_ + W

    lax.while_loop(cond, body, (gmin // 8) * 8)

    @pl.when(pid == nb - 1)
    def _():
        sums = acc[0:G, :]
        c = cnt[0:G, 0:1]
        means = sums / jnp.maximum(c, 1.0)
        q = _elu(jnp.dot(means, gw0_ref[...], preferred_element_type=f32)
                 + gb0_ref[...])
        q = _elu(jnp.dot(q, gw1_ref[...], preferred_element_type=f32)
                 + gb1_ref[...])
        qjt_ref[...] = (jnp.dot(q, gw2_ref[...], preferred_element_type=f32)
                        + gb2_ref[...])
        means_ref[...] = means


def _phase_e_body(enc_ref, key1_ref, kbar_ref, pw0_ref, pb0_ref, pw1_ref,
                  pb1_ref, out_ref):
    d = enc_ref[...] + kbar_ref[...] - key1_ref[...]
    pre = (jnp.dot(d, pw0_ref[...], preferred_element_type=jnp.float32)
           + pb0_ref[...])
    out_ref[...] = (jnp.dot(_elu(pre), pw1_ref[...],
                            preferred_element_type=jnp.float32) + pb1_ref[...])


def kernel(flat_pair_enc, flat_pair_encA, group_index,
           phi1_W0, phi1_b0, phi1_W1, phi1_b1, phi1_W2, phi1_b2,
           g_W0, g_b0, g_W1, g_b1, g_W2, g_b2,
           phi2_W0, phi2_b0, phi2_W1, phi2_b1):
    N, DA = flat_pair_encA.shape
    H = phi1_W1.shape[0]
    D2 = 2 * H
    f32 = jnp.float32
    i32 = jnp.int32

    info = plsc.get_sparse_core_info()
    NC, NS = info.num_cores, info.num_subcores
    NW = NC * NS
    NBLK = N // RB
    T = -(-NBLK // NW)
    NB = N // BA

    b = lambda v: v.reshape(1, -1)
    gi3 = group_index.reshape(NB, 1, BA)

    key1, qjt, means = pl.pallas_call(
        _phase_abc_body,
        grid=(NB,),
        in_specs=[
            pl.BlockSpec((BA, DA), lambda i: (i, 0)),
            pl.BlockSpec((1, 1, BA), lambda i: (i, 0, 0)),
            pl.BlockSpec((DA, H), lambda i: (0, 0)),
            pl.BlockSpec((1, H), lambda i: (0, 0)),
            pl.BlockSpec((H, H), lambda i: (0, 0)),
            pl.BlockSpec((1, H), lambda i: (0, 0)),
            pl.BlockSpec((H, D2), lambda i: (0, 0)),
            pl.BlockSpec((1, D2), lambda i: (0, 0)),
            pl.BlockSpec((D2, H), lambda i: (0, 0)),
            pl.BlockSpec((1, H), lambda i: (0, 0)),
            pl.BlockSpec((H, H), lambda i: (0, 0)),
            pl.BlockSpec((1, H), lambda i: (0, 0)),
            pl.BlockSpec((H, 1), lambda i: (0, 0)),
            pl.BlockSpec((1, 1), lambda i: (0, 0)),
        ],
        out_specs=[
            pl.BlockSpec((BA, D2), lambda i: (i, 0)),
            pl.BlockSpec((G, 1), lambda i: (0, 0)),
            pl.BlockSpec((G, D2), lambda i: (0, 0)),
        ],
        out_shape=[
            jax.ShapeDtypeStruct((N, D2), f32),
            jax.ShapeDtypeStruct((G, 1), f32),
            jax.ShapeDtypeStruct((G, D2), f32),
        ],
        scratch_shapes=[
            pltpu.VMEM((GA, D2), f32),
            pltpu.VMEM((GA, 8), f32),
        ],
    )(flat_pair_encA, gi3, phi1_W0, b(phi1_b0), phi1_W1, b(phi1_b1),
      phi1_W2, b(phi1_b2), g_W0, b(g_b0), g_W1, b(g_b1), g_W2,
      g_b2.reshape(1, 1))

    mesh = plsc.VectorSubcoreMesh(core_axis_name="c", subcore_axis_name="s")

    @functools.partial(
        pl.kernel, mesh=mesh,
        out_type=jax.ShapeDtypeStruct((N, D2), f32),
        scratch_types=[
            pltpu.VMEM((RB,), i32),
            pltpu.VMEM((RB, D2), f32),
            pltpu.SemaphoreType.DMA,
        ],
    )
    def _sc_gather(tab_hbm, gi_hbm, out_hbm, idx_v, rows_v, sem):
        cid = lax.axis_index("c")
        sid = lax.axis_index("s")
        wid = sid * NC + cid

        def trip(t, cc):
            blk = t * NW + wid

            @pl.when(blk < NBLK)
            def _():
                r0 = blk * RB
                pltpu.sync_copy(gi_hbm.at[pl.ds(r0, RB)], idx_v)
                pltpu.async_copy(tab_hbm.at[idx_v], rows_v, sem).wait()
                pltpu.sync_copy(rows_v, out_hbm.at[pl.ds(r0, RB)])
            return cc

        lax.fori_loop(0, T, trip, 0)

    kbar = _sc_gather(means, group_index)

    altq = pl.pallas_call(
        _phase_e_body,
        grid=(N // BE,),
        in_specs=[
            pl.BlockSpec((BE, D2), lambda i: (i, 0)),
            pl.BlockSpec((BE, D2), lambda i: (i, 0)),
            pl.BlockSpec((BE, D2), lambda i: (i, 0)),
            pl.BlockSpec((D2, H), lambda i: (0, 0)),
            pl.BlockSpec((1, H), lambda i: (0, 0)),
            pl.BlockSpec((H, 2), lambda i: (0, 0)),
            pl.BlockSpec((1, 2), lambda i: (0, 0)),
        ],
        out_specs=pl.BlockSpec((BE, 2), lambda i: (i, 0)),
        out_shape=jax.ShapeDtypeStruct((N, 2), f32),
    )(flat_pair_enc, key1, kbar, phi2_W0, b(phi2_b0), phi2_W1, b(phi2_b1))

    return (qjt, altq)


# speedup vs baseline: 2.4312x; 1.0919x over previous
import functools

import jax
import jax.numpy as jnp
from jax import lax
from jax.experimental import pallas as pl
from jax.experimental.pallas import tpu as pltpu
from jax.experimental.pallas import tpu_sc as plsc

G = 10000
GA = 10624
W = 128
BA = 512
RB = 128
BE = 1280


def _elu(x):
    return jnp.where(x > 0, x, jnp.exp(jnp.minimum(x, 0.0)) - 1.0)


def _phase_abc_body(enca_ref, gi_ref, w0_ref, b0_ref, w1_ref, b1_ref,
                    w2_ref, b2_ref, gw0_ref, gb0_ref, gw1_ref, gb1_ref,
                    gw2_ref, gb2_ref, key1_ref, qjt_ref, means_ref,
                    acc, cnt):
    f32 = jnp.float32
    pid = pl.program_id(0)
    nb = pl.num_programs(0)

    @pl.when(pid == 0)
    def _():
        acc[...] = jnp.zeros(acc.shape, f32)
        cnt[...] = jnp.zeros(cnt.shape, f32)

    x = enca_ref[...]
    h = _elu(jnp.dot(x, w0_ref[...], preferred_element_type=f32)
             + b0_ref[...])
    h = _elu(jnp.dot(h, w1_ref[...], preferred_element_type=f32)
             + b1_ref[...])
    k1 = jnp.dot(h, w2_ref[...], preferred_element_type=f32) + b2_ref[...]
    key1_ref[...] = k1

    gi_row = gi_ref[0]
    gmin = jnp.min(gi_row)
    gmax = jnp.max(gi_row)
    onesb = jnp.ones((BA, 8), f32)

    def cond(b_):
        return b_ <= gmax

    def body(b_):
        b8 = pl.multiple_of(b_, 8)
        ids = b8 + lax.broadcasted_iota(jnp.int32, (W, 1), 0)
        oh = (ids == gi_row).astype(f32)
        part = jnp.dot(oh, k1, preferred_element_type=f32)
        pc = jnp.dot(oh, onesb, preferred_element_type=f32)
        acc[pl.ds(b8, W), :] = acc[pl.ds(b8, W), :] + part
        cnt[pl.ds(b8, W), :] = cnt[pl.ds(b8, W), :] + pc
        return b_ + W

    lax.while_loop(cond, body, (gmin // 8) * 8)

    @pl.when(pid == nb - 1)
    def _():
        sums = acc[0:G, :]
        c = cnt[0:G, 0:1]
        means = sums / jnp.maximum(c, 1.0)
        q = _elu(jnp.dot(means, gw0_ref[...], preferred_element_type=f32)
                 + gb0_ref[...])
        q = _elu(jnp.dot(q, gw1_ref[...], preferred_element_type=f32)
                 + gb1_ref[...])
        qjt_ref[...] = (jnp.dot(q, gw2_ref[...], preferred_element_type=f32)
                        + gb2_ref[...])
        means_ref[...] = means


def _phase_e_body(enc_ref, key1_ref, kbar_ref, pw0_ref, pb0_ref, pw1_ref,
                  pb1_ref, out_ref):
    d = enc_ref[...] + kbar_ref[...] - key1_ref[...]
    pre = (jnp.dot(d, pw0_ref[...], preferred_element_type=jnp.float32) + pb0_ref[...])
    out_ref[...] = (jnp.dot(_elu(pre), pw1_ref[...],
                            preferred_element_type=jnp.float32) + pb1_ref[...])


def kernel(flat_pair_enc, flat_pair_encA, group_index,
           phi1_W0, phi1_b0, phi1_W1, phi1_b1, phi1_W2, phi1_b2,
           g_W0, g_b0, g_W1, g_b1, g_W2, g_b2,
           phi2_W0, phi2_b0, phi2_W1, phi2_b1):
    N, DA = flat_pair_encA.shape
    H = phi1_W1.shape[0]
    D2 = 2 * H
    f32 = jnp.float32
    i32 = jnp.int32

    info = plsc.get_sparse_core_info()
    NC, NS = info.num_cores, info.num_subcores
    NW = NC * NS
    NBLK = N // RB
    T = -(-NBLK // NW)
    NB = N // BA

    b = lambda v: v.reshape(1, -1)
    gi3 = group_index.reshape(NB, 1, BA)

    key1, qjt, means = pl.pallas_call(
        _phase_abc_body,
        grid=(NB,),
        in_specs=[
            pl.BlockSpec((BA, DA), lambda i: (i, 0)),
            pl.BlockSpec((1, 1, BA), lambda i: (i, 0, 0)),
            pl.BlockSpec((DA, H), lambda i: (0, 0)),
            pl.BlockSpec((1, H), lambda i: (0, 0)),
            pl.BlockSpec((H, H), lambda i: (0, 0)),
            pl.BlockSpec((1, H), lambda i: (0, 0)),
            pl.BlockSpec((H, D2), lambda i: (0, 0)),
            pl.BlockSpec((1, D2), lambda i: (0, 0)),
            pl.BlockSpec((D2, H), lambda i: (0, 0)),
            pl.BlockSpec((1, H), lambda i: (0, 0)),
            pl.BlockSpec((H, H), lambda i: (0, 0)),
            pl.BlockSpec((1, H), lambda i: (0, 0)),
            pl.BlockSpec((H, 1), lambda i: (0, 0)),
            pl.BlockSpec((1, 1), lambda i: (0, 0)),
        ],
        out_specs=[
            pl.BlockSpec((BA, D2), lambda i: (i, 0)),
            pl.BlockSpec((G, 1), lambda i: (0, 0)),
            pl.BlockSpec((G, D2), lambda i: (0, 0)),
        ],
        out_shape=[
            jax.ShapeDtypeStruct((N, D2), f32),
            jax.ShapeDtypeStruct((G, 1), f32),
            jax.ShapeDtypeStruct((G, D2), f32),
        ],
        scratch_shapes=[
            pltpu.VMEM((GA, D2), f32),
            pltpu.VMEM((GA, 8), f32),
        ],
    )(flat_pair_encA, gi3, phi1_W0, b(phi1_b0), phi1_W1, b(phi1_b1),
      phi1_W2, b(phi1_b2), g_W0, b(g_b0), g_W1, b(g_b1), g_W2,
      g_b2.reshape(1, 1))

    mesh = plsc.VectorSubcoreMesh(core_axis_name="c", subcore_axis_name="s")

    SB = 4 * RB
    NSB = N // SB
    TSB = -(-NSB // NW)

    @functools.partial(
        pl.kernel, mesh=mesh,
        out_type=jax.ShapeDtypeStruct((N, D2), f32),
        scratch_types=[
            pltpu.VMEM((SB,), i32),
            pltpu.VMEM((SB, D2), f32),
            pltpu.SemaphoreType.DMA,
        ],
    )
    def _sc_gather(tab_hbm, gi_hbm, out_hbm, idx_v, rows_v, sem):
        cid = lax.axis_index("c")
        sid = lax.axis_index("s")
        wid = sid * NC + cid

        def trip(t, cc):
            sb = t * NW + wid

            @pl.when(sb < NSB)
            def _():
                r0 = sb * SB
                pltpu.sync_copy(gi_hbm.at[pl.ds(r0, SB)], idx_v)
                cps = [
                    pltpu.async_copy(
                        tab_hbm.at[idx_v.at[pl.ds(k * RB, RB)]],
                        rows_v.at[pl.ds(k * RB, RB)], sem)
                    for k in range(SB // RB)
                ]
                for cp in cps:
                    cp.wait()
                pltpu.sync_copy(rows_v, out_hbm.at[pl.ds(r0, SB)])
            return cc

        lax.fori_loop(0, TSB, trip, 0)

    kbar = _sc_gather(means, group_index)

    altq = pl.pallas_call(
        _phase_e_body,
        grid=(N // BE,),
        in_specs=[
            pl.BlockSpec((BE, D2), lambda i: (i, 0)),
            pl.BlockSpec((BE, D2), lambda i: (i, 0)),
            pl.BlockSpec((BE, D2), lambda i: (i, 0)),
            pl.BlockSpec((D2, H), lambda i: (0, 0)),
            pl.BlockSpec((1, H), lambda i: (0, 0)),
            pl.BlockSpec((H, 2), lambda i: (0, 0)),
            pl.BlockSpec((1, 2), lambda i: (0, 0)),
        ],
        out_specs=pl.BlockSpec((BE, 2), lambda i: (i, 0)),
        out_shape=jax.ShapeDtypeStruct((N, 2), f32),
    )(flat_pair_enc, key1, kbar, phi2_W0, b(phi2_b0), phi2_W1, b(phi2_b1))

    return (qjt, altq)


# speedup vs baseline: 3.1293x; 1.2871x over previous
import functools

import jax
import jax.numpy as jnp
from jax import lax
from jax.experimental import pallas as pl
from jax.experimental.pallas import tpu as pltpu
from jax.experimental.pallas import tpu_sc as plsc

G = 10000
GA = 10624
W = 128
BA = 1280
RB = 128
BE = 2560


def _elu(x):
    return jnp.where(x > 0, x, jnp.exp(jnp.minimum(x, 0.0)) - 1.0)


def _phase_abc_body(enca_ref, gi_ref, w0_ref, b0_ref, w1_ref, b1_ref,
                    w2_ref, b2_ref, gw0_ref, gb0_ref, gw1_ref, gb1_ref,
                    gw2_ref, gb2_ref, key1_ref, qjt_ref, means_ref,
                    acc, cnt):
    f32 = jnp.float32
    pid = pl.program_id(0)
    nb = pl.num_programs(0)

    @pl.when(pid == 0)
    def _():
        acc[...] = jnp.zeros(acc.shape, f32)
        cnt[...] = jnp.zeros(cnt.shape, f32)

    x = enca_ref[...]
    h = _elu(jnp.dot(x, w0_ref[...], preferred_element_type=f32)
             + b0_ref[...])
    h = _elu(jnp.dot(h, w1_ref[...], preferred_element_type=f32)
             + b1_ref[...])
    k1 = jnp.dot(h, w2_ref[...], preferred_element_type=f32) + b2_ref[...]
    key1_ref[...] = k1

    gi_row = gi_ref[0]
    gmin = jnp.min(gi_row)
    gmax = jnp.max(gi_row)
    onesb = jnp.ones((BA, 8), f32)

    def cond(b_):
        return b_ <= gmax

    def body(b_):
        b8 = pl.multiple_of(b_, 8)
        ids = b8 + lax.broadcasted_iota(jnp.int32, (W, 1), 0)
        oh = (ids == gi_row).astype(f32)
        part = jnp.dot(oh, k1, preferred_element_type=f32)
        pc = jnp.dot(oh, onesb, preferred_element_type=f32)
        acc[pl.ds(b8, W), :] = acc[pl.ds(b8, W), :] + part
        cnt[pl.ds(b8, W), :] = cnt[pl.ds(b8, W), :] + pc
        return b_ + W

    lax.while_loop(cond, body, (gmin // 8) * 8)

    @pl.when(pid == nb - 1)
    def _():
        sums = acc[0:G, :]
        c = cnt[0:G, 0:1]
        means = sums / jnp.maximum(c, 1.0)
        q = _elu(jnp.dot(means, gw0_ref[...], preferred_element_type=f32)
                 + gb0_ref[...])
        q = _elu(jnp.dot(q, gw1_ref[...], preferred_element_type=f32)
                 + gb1_ref[...])
        qjt_ref[...] = (jnp.dot(q, gw2_ref[...], preferred_element_type=f32)
                        + gb2_ref[...])
        means_ref[...] = means


def _phase_e_body(enc_ref, key1_ref, kbar_ref, pw0_ref, pb0_ref, pw1_ref,
                  pb1_ref, out_ref):
    d = enc_ref[...] + kbar_ref[...] - key1_ref[...]
    pre = (jnp.dot(d, pw0_ref[...], preferred_element_type=jnp.float32) + pb0_ref[...])
    out_ref[...] = (jnp.dot(_elu(pre), pw1_ref[...],
                            preferred_element_type=jnp.float32) + pb1_ref[...])


def kernel(flat_pair_enc, flat_pair_encA, group_index,
           phi1_W0, phi1_b0, phi1_W1, phi1_b1, phi1_W2, phi1_b2,
           g_W0, g_b0, g_W1, g_b1, g_W2, g_b2,
           phi2_W0, phi2_b0, phi2_W1, phi2_b1):
    N, DA = flat_pair_encA.shape
    H = phi1_W1.shape[0]
    D2 = 2 * H
    f32 = jnp.float32
    i32 = jnp.int32

    info = plsc.get_sparse_core_info()
    NC, NS = info.num_cores, info.num_subcores
    NW = NC * NS
    NBLK = N // RB
    T = -(-NBLK // NW)
    NB = N // BA

    b = lambda v: v.reshape(1, -1)
    gi3 = group_index.reshape(NB, 1, BA)

    key1, qjt, means = pl.pallas_call(
        _phase_abc_body,
        grid=(NB,),
        in_specs=[
            pl.BlockSpec((BA, DA), lambda i: (i, 0)),
            pl.BlockSpec((1, 1, BA), lambda i: (i, 0, 0)),
            pl.BlockSpec((DA, H), lambda i: (0, 0)),
            pl.BlockSpec((1, H), lambda i: (0, 0)),
            pl.BlockSpec((H, H), lambda i: (0, 0)),
            pl.BlockSpec((1, H), lambda i: (0, 0)),
            pl.BlockSpec((H, D2), lambda i: (0, 0)),
            pl.BlockSpec((1, D2), lambda i: (0, 0)),
            pl.BlockSpec((D2, H), lambda i: (0, 0)),
            pl.BlockSpec((1, H), lambda i: (0, 0)),
            pl.BlockSpec((H, H), lambda i: (0, 0)),
            pl.BlockSpec((1, H), lambda i: (0, 0)),
            pl.BlockSpec((H, 1), lambda i: (0, 0)),
            pl.BlockSpec((1, 1), lambda i: (0, 0)),
        ],
        out_specs=[
            pl.BlockSpec((BA, D2), lambda i: (i, 0)),
            pl.BlockSpec((G, 1), lambda i: (0, 0)),
            pl.BlockSpec((G, D2), lambda i: (0, 0)),
        ],
        out_shape=[
            jax.ShapeDtypeStruct((N, D2), f32),
            jax.ShapeDtypeStruct((G, 1), f32),
            jax.ShapeDtypeStruct((G, D2), f32),
        ],
        scratch_shapes=[
            pltpu.VMEM((GA, D2), f32),
            pltpu.VMEM((GA, 8), f32),
        ],
    )(flat_pair_encA, gi3, phi1_W0, b(phi1_b0), phi1_W1, b(phi1_b1),
      phi1_W2, b(phi1_b2), g_W0, b(g_b0), g_W1, b(g_b1), g_W2,
      g_b2.reshape(1, 1))

    mesh = plsc.VectorSubcoreMesh(core_axis_name="c", subcore_axis_name="s")

    SB = 4 * RB
    NSB = N // SB
    TSB = -(-NSB // NW)

    @functools.partial(
        pl.kernel, mesh=mesh,
        out_type=jax.ShapeDtypeStruct((N, D2), f32),
        scratch_types=[
            pltpu.VMEM((SB,), i32),
            pltpu.VMEM((SB, D2), f32),
            pltpu.SemaphoreType.DMA,
        ],
    )
    def _sc_gather(tab_hbm, gi_hbm, out_hbm, idx_v, rows_v, sem):
        cid = lax.axis_index("c")
        sid = lax.axis_index("s")
        wid = sid * NC + cid

        def trip(t, cc):
            sb = t * NW + wid

            @pl.when(sb < NSB)
            def _():
                r0 = sb * SB
                pltpu.sync_copy(gi_hbm.at[pl.ds(r0, SB)], idx_v)
                cps = [
                    pltpu.async_copy(
                        tab_hbm.at[idx_v.at[pl.ds(k * RB, RB)]],
                        rows_v.at[pl.ds(k * RB, RB)], sem)
                    for k in range(SB // RB)
                ]
                for cp in cps:
                    cp.wait()
                pltpu.sync_copy(rows_v, out_hbm.at[pl.ds(r0, SB)])
            return cc

        lax.fori_loop(0, TSB, trip, 0)

    kbar = _sc_gather(means, group_index)

    altq = pl.pallas_call(
        _phase_e_body,
        grid=(N // BE,),
        in_specs=[
            pl.BlockSpec((BE, D2), lambda i: (i, 0)),
            pl.BlockSpec((BE, D2), lambda i: (i, 0)),
            pl.BlockSpec((BE, D2), lambda i: (i, 0)),
            pl.BlockSpec((D2, H), lambda i: (0, 0)),
            pl.BlockSpec((1, H), lambda i: (0, 0)),
            pl.BlockSpec((H, 2), lambda i: (0, 0)),
            pl.BlockSpec((1, 2), lambda i: (0, 0)),
        ],
        out_specs=pl.BlockSpec((BE, 2), lambda i: (i, 0)),
        out_shape=jax.ShapeDtypeStruct((N, 2), f32),
    )(flat_pair_enc, key1, kbar, phi2_W0, b(phi2_b0), phi2_W1, b(phi2_b1))

    return (qjt, altq)


# speedup vs baseline: 3.4335x; 1.0972x over previous
import functools

import jax
import jax.numpy as jnp
from jax import lax
from jax.experimental import pallas as pl
from jax.experimental.pallas import tpu as pltpu
from jax.experimental.pallas import tpu_sc as plsc

G = 10000
GA = 10624
W = 128
BA = 2560
RB = 128
BE = 6400


def _elu(x):
    return jnp.where(x > 0, x, jnp.exp(jnp.minimum(x, 0.0)) - 1.0)


def _phase_abc_body(enca_ref, gi_ref, w0_ref, b0_ref, w1_ref, b1_ref,
                    w2_ref, b2_ref, gw0_ref, gb0_ref, gw1_ref, gb1_ref,
                    gw2_ref, gb2_ref, key1_ref, qjt_ref, means_ref,
                    acc, cnt):
    f32 = jnp.float32
    pid = pl.program_id(0)
    nb = pl.num_programs(0)

    @pl.when(pid == 0)
    def _():
        acc[...] = jnp.zeros(acc.shape, f32)
        cnt[...] = jnp.zeros(cnt.shape, f32)

    x = enca_ref[...]
    h = _elu(jnp.dot(x, w0_ref[...], preferred_element_type=f32)
             + b0_ref[...])
    h = _elu(jnp.dot(h, w1_ref[...], preferred_element_type=f32)
             + b1_ref[...])
    k1 = jnp.dot(h, w2_ref[...], preferred_element_type=f32) + b2_ref[...]
    key1_ref[...] = k1

    gi_row = gi_ref[0]
    gmin = jnp.min(gi_row)
    gmax = jnp.max(gi_row)
    onesb = jnp.ones((BA, 8), f32)

    def cond(b_):
        return b_ <= gmax

    def body(b_):
        b8 = pl.multiple_of(b_, 8)
        ids = b8 + lax.broadcasted_iota(jnp.int32, (W, 1), 0)
        oh = (ids == gi_row).astype(f32)
        part = jnp.dot(oh, k1, preferred_element_type=f32)
        pc = jnp.dot(oh, onesb, preferred_element_type=f32)
        acc[pl.ds(b8, W), :] = acc[pl.ds(b8, W), :] + part
        cnt[pl.ds(b8, W), :] = cnt[pl.ds(b8, W), :] + pc
        return b_ + W

    lax.while_loop(cond, body, (gmin // 8) * 8)

    @pl.when(pid == nb - 1)
    def _():
        sums = acc[0:G, :]
        c = cnt[0:G, 0:1]
        means = sums / jnp.maximum(c, 1.0)
        q = _elu(jnp.dot(means, gw0_ref[...], preferred_element_type=f32)
                 + gb0_ref[...])
        q = _elu(jnp.dot(q, gw1_ref[...], preferred_element_type=f32)
                 + gb1_ref[...])
        qjt_ref[...] = (jnp.dot(q, gw2_ref[...], preferred_element_type=f32)
                        + gb2_ref[...])
        means_ref[...] = means


def _phase_e_body(enc_ref, key1_ref, kbar_ref, pw0_ref, pb0_ref, pw1_ref,
                  pb1_ref, out_ref):
    d = enc_ref[...] + kbar_ref[...] - key1_ref[...]
    pre = (jnp.dot(d, pw0_ref[...], preferred_element_type=jnp.float32) + pb0_ref[...])
    out_ref[...] = (jnp.dot(_elu(pre), pw1_ref[...],
                            preferred_element_type=jnp.float32) + pb1_ref[...])


def kernel(flat_pair_enc, flat_pair_encA, group_index,
           phi1_W0, phi1_b0, phi1_W1, phi1_b1, phi1_W2, phi1_b2,
           g_W0, g_b0, g_W1, g_b1, g_W2, g_b2,
           phi2_W0, phi2_b0, phi2_W1, phi2_b1):
    N, DA = flat_pair_encA.shape
    H = phi1_W1.shape[0]
    D2 = 2 * H
    f32 = jnp.float32
    i32 = jnp.int32

    info = plsc.get_sparse_core_info()
    NC, NS = info.num_cores, info.num_subcores
    NW = NC * NS
    NBLK = N // RB
    T = -(-NBLK // NW)
    NB = N // BA

    b = lambda v: v.reshape(1, -1)
    gi3 = group_index.reshape(NB, 1, BA)

    key1, qjt, means = pl.pallas_call(
        _phase_abc_body,
        grid=(NB,),
        in_specs=[
            pl.BlockSpec((BA, DA), lambda i: (i, 0)),
            pl.BlockSpec((1, 1, BA), lambda i: (i, 0, 0)),
            pl.BlockSpec((DA, H), lambda i: (0, 0)),
            pl.BlockSpec((1, H), lambda i: (0, 0)),
            pl.BlockSpec((H, H), lambda i: (0, 0)),
            pl.BlockSpec((1, H), lambda i: (0, 0)),
            pl.BlockSpec((H, D2), lambda i: (0, 0)),
            pl.BlockSpec((1, D2), lambda i: (0, 0)),
            pl.BlockSpec((D2, H), lambda i: (0, 0)),
            pl.BlockSpec((1, H), lambda i: (0, 0)),
            pl.BlockSpec((H, H), lambda i: (0, 0)),
            pl.BlockSpec((1, H), lambda i: (0, 0)),
            pl.BlockSpec((H, 1), lambda i: (0, 0)),
            pl.BlockSpec((1, 1), lambda i: (0, 0)),
        ],
        out_specs=[
            pl.BlockSpec((BA, D2), lambda i: (i, 0)),
            pl.BlockSpec((G, 1), lambda i: (0, 0)),
            pl.BlockSpec((G, D2), lambda i: (0, 0)),
        ],
        out_shape=[
            jax.ShapeDtypeStruct((N, D2), f32),
            jax.ShapeDtypeStruct((G, 1), f32),
            jax.ShapeDtypeStruct((G, D2), f32),
        ],
        scratch_shapes=[
            pltpu.VMEM((GA, D2), f32),
            pltpu.VMEM((GA, 8), f32),
        ],
    )(flat_pair_encA, gi3, phi1_W0, b(phi1_b0), phi1_W1, b(phi1_b1),
      phi1_W2, b(phi1_b2), g_W0, b(g_b0), g_W1, b(g_b1), g_W2,
      g_b2.reshape(1, 1))

    mesh = plsc.VectorSubcoreMesh(core_axis_name="c", subcore_axis_name="s")

    SB = 4 * RB
    NSB = N // SB
    TSB = -(-NSB // NW)

    @functools.partial(
        pl.kernel, mesh=mesh,
        out_type=jax.ShapeDtypeStruct((N, D2), f32),
        scratch_types=[
            pltpu.VMEM((SB,), i32),
            pltpu.VMEM((SB, D2), f32),
            pltpu.SemaphoreType.DMA,
        ],
    )
    def _sc_gather(tab_hbm, gi_hbm, out_hbm, idx_v, rows_v, sem):
        cid = lax.axis_index("c")
        sid = lax.axis_index("s")
        wid = sid * NC + cid

        def trip(t, cc):
            sb = t * NW + wid

            @pl.when(sb < NSB)
            def _():
                r0 = sb * SB
                pltpu.sync_copy(gi_hbm.at[pl.ds(r0, SB)], idx_v)
                cps = [
                    pltpu.async_copy(
                        tab_hbm.at[idx_v.at[pl.ds(k * RB, RB)]],
                        rows_v.at[pl.ds(k * RB, RB)], sem)
                    for k in range(SB // RB)
                ]
                for cp in cps:
                    cp.wait()
                pltpu.sync_copy(rows_v, out_hbm.at[pl.ds(r0, SB)])
            return cc

        lax.fori_loop(0, TSB, trip, 0)

    kbar = _sc_gather(means, group_index)

    altq = pl.pallas_call(
        _phase_e_body,
        grid=(N // BE,),
        in_specs=[
            pl.BlockSpec((BE, D2), lambda i: (i, 0)),
            pl.BlockSpec((BE, D2), lambda i: (i, 0)),
            pl.BlockSpec((BE, D2), lambda i: (i, 0)),
            pl.BlockSpec((D2, H), lambda i: (0, 0)),
            pl.BlockSpec((1, H), lambda i: (0, 0)),
            pl.BlockSpec((H, 2), lambda i: (0, 0)),
            pl.BlockSpec((1, 2), lambda i: (0, 0)),
        ],
        out_specs=pl.BlockSpec((BE, 2), lambda i: (i, 0)),
        out_shape=jax.ShapeDtypeStruct((N, 2), f32),
    )(flat_pair_enc, key1, kbar, phi2_W0, b(phi2_b0), phi2_W1, b(phi2_b1))

    return (qjt, altq)


# speedup vs baseline: 3.4515x; 1.0052x over previous
import functools

import jax
import jax.numpy as jnp
from jax import lax
from jax.experimental import pallas as pl
from jax.experimental.pallas import tpu as pltpu
from jax.experimental.pallas import tpu_sc as plsc

G = 10000
GA = 10624
W = 128
BA = 2560
RB = 128
BE = 6400


def _elu(x):
    return jnp.where(x > 0, x, jnp.exp(jnp.minimum(x, 0.0)) - 1.0)


def _phase_abc_body(enca_ref, gi_ref, w0_ref, b0_ref, w1_ref, b1_ref,
                    w2_ref, b2_ref, gw0_ref, gb0_ref, gw1_ref, gb1_ref,
                    gw2_ref, gb2_ref, key1_ref, qjt_ref, means_ref,
                    acc, cnt):
    f32 = jnp.float32
    pid = pl.program_id(0)
    nb = pl.num_programs(0)

    @pl.when(pid == 0)
    def _():
        acc[...] = jnp.zeros(acc.shape, f32)
        cnt[...] = jnp.zeros(cnt.shape, f32)

    x = enca_ref[...]
    h = _elu(jnp.dot(x, w0_ref[...], preferred_element_type=f32)
             + b0_ref[...])
    h = _elu(jnp.dot(h, w1_ref[...], preferred_element_type=f32)
             + b1_ref[...])
    k1 = jnp.dot(h, w2_ref[...], preferred_element_type=f32) + b2_ref[...]
    key1_ref[...] = k1

    gi_row = gi_ref[0]
    gmin = jnp.min(gi_row)
    gmax = jnp.max(gi_row)
    onesb = jnp.ones((BA, 8), f32)

    def cond(b_):
        return b_ <= gmax

    def body(b_):
        b8 = pl.multiple_of(b_, 8)
        ids = b8 + lax.broadcasted_iota(jnp.int32, (W, 1), 0)
        oh = (ids == gi_row).astype(f32)
        part = jnp.dot(oh, k1, preferred_element_type=f32)
        pc = jnp.dot(oh, onesb, preferred_element_type=f32)
        acc[pl.ds(b8, W), :] = acc[pl.ds(b8, W), :] + part
        cnt[pl.ds(b8, W), :] = cnt[pl.ds(b8, W), :] + pc
        return b_ + W

    lax.while_loop(cond, body, (gmin // 8) * 8)

    @pl.when(pid == nb - 1)
    def _():
        sums = acc[0:G, :]
        c = cnt[0:G, 0:1]
        means = sums / jnp.maximum(c, 1.0)
        q = _elu(jnp.dot(means, gw0_ref[...], preferred_element_type=f32)
                 + gb0_ref[...])
        q = _elu(jnp.dot(q, gw1_ref[...], preferred_element_type=f32)
                 + gb1_ref[...])
        qjt_ref[...] = (jnp.dot(q, gw2_ref[...], preferred_element_type=f32)
                        + gb2_ref[...])
        means_ref[...] = means


def _phase_e_body(enc_ref, key1_ref, kbar_ref, pw0_ref, pb0_ref, pw1_ref,
                  pb1_ref, out_ref):
    d = enc_ref[...] + kbar_ref[...] - key1_ref[...]
    pre = (jnp.dot(d, pw0_ref[...], preferred_element_type=jnp.float32) + pb0_ref[...])
    out_ref[...] = (jnp.dot(_elu(pre), pw1_ref[...],
                            preferred_element_type=jnp.float32) + pb1_ref[...])


def kernel(flat_pair_enc, flat_pair_encA, group_index,
           phi1_W0, phi1_b0, phi1_W1, phi1_b1, phi1_W2, phi1_b2,
           g_W0, g_b0, g_W1, g_b1, g_W2, g_b2,
           phi2_W0, phi2_b0, phi2_W1, phi2_b1):
    N, DA = flat_pair_encA.shape
    H = phi1_W1.shape[0]
    D2 = 2 * H
    f32 = jnp.float32
    i32 = jnp.int32

    info = plsc.get_sparse_core_info()
    NC, NS = info.num_cores, info.num_subcores
    NW = NC * NS
    NBLK = N // RB
    T = -(-NBLK // NW)
    NB = N // BA

    b = lambda v: v.reshape(1, -1)
    gi3 = group_index.reshape(NB, 1, BA)

    key1, qjt, means = pl.pallas_call(
        _phase_abc_body,
        grid=(NB,),
        in_specs=[
            pl.BlockSpec((BA, DA), lambda i: (i, 0)),
            pl.BlockSpec((1, 1, BA), lambda i: (i, 0, 0)),
            pl.BlockSpec((DA, H), lambda i: (0, 0)),
            pl.BlockSpec((1, H), lambda i: (0, 0)),
            pl.BlockSpec((H, H), lambda i: (0, 0)),
            pl.BlockSpec((1, H), lambda i: (0, 0)),
            pl.BlockSpec((H, D2), lambda i: (0, 0)),
            pl.BlockSpec((1, D2), lambda i: (0, 0)),
            pl.BlockSpec((D2, H), lambda i: (0, 0)),
            pl.BlockSpec((1, H), lambda i: (0, 0)),
            pl.BlockSpec((H, H), lambda i: (0, 0)),
            pl.BlockSpec((1, H), lambda i: (0, 0)),
            pl.BlockSpec((H, 1), lambda i: (0, 0)),
            pl.BlockSpec((1, 1), lambda i: (0, 0)),
        ],
        out_specs=[
            pl.BlockSpec((BA, D2), lambda i: (i, 0)),
            pl.BlockSpec((G, 1), lambda i: (0, 0)),
            pl.BlockSpec((G, D2), lambda i: (0, 0)),
        ],
        out_shape=[
            jax.ShapeDtypeStruct((N, D2), f32),
            jax.ShapeDtypeStruct((G, 1), f32),
            jax.ShapeDtypeStruct((G, D2), f32),
        ],
        scratch_shapes=[
            pltpu.VMEM((GA, D2), f32),
            pltpu.VMEM((GA, 8), f32),
        ],
    )(flat_pair_encA, gi3, phi1_W0, b(phi1_b0), phi1_W1, b(phi1_b1),
      phi1_W2, b(phi1_b2), g_W0, b(g_b0), g_W1, b(g_b1), g_W2,
      g_b2.reshape(1, 1))

    mesh = plsc.VectorSubcoreMesh(core_axis_name="c", subcore_axis_name="s")

    SB = 4 * RB
    NSBH = (N // 2) // SB
    TSB = -(-NSBH // NW)

    def _make_gather(base):
        @functools.partial(
            pl.kernel, mesh=mesh,
            out_type=jax.ShapeDtypeStruct((N // 2, D2), f32),
            scratch_types=[
                pltpu.VMEM((SB,), i32),
                pltpu.VMEM((SB, D2), f32),
                pltpu.SemaphoreType.DMA,
            ],
        )
        def _sc_gather(tab_hbm, gi_hbm, out_hbm, idx_v, rows_v, sem):
            cid = lax.axis_index("c")
            sid = lax.axis_index("s")
            wid = sid * NC + cid

            def trip(t, cc):
                sb = t * NW + wid

                @pl.when(sb < NSBH)
                def _():
                    r0 = sb * SB
                    pltpu.sync_copy(gi_hbm.at[pl.ds(base + r0, SB)], idx_v)
                    cps = [
                        pltpu.async_copy(
                            tab_hbm.at[idx_v.at[pl.ds(k * RB, RB)]],
                            rows_v.at[pl.ds(k * RB, RB)], sem)
                        for k in range(SB // RB)
                    ]
                    for cp in cps:
                        cp.wait()
                    pltpu.sync_copy(rows_v, out_hbm.at[pl.ds(r0, SB)])
                return cc

            lax.fori_loop(0, TSB, trip, 0)

        return _sc_gather

    kbar0 = _make_gather(0)(means, group_index)
    kbar1 = _make_gather(N // 2)(means, group_index)

    NEH = (N // 2) // BE

    def _run_e(kbar_h, half):
        off = half * NEH
        return pl.pallas_call(
            _phase_e_body,
            grid=(NEH,),
            in_specs=[
                pl.BlockSpec((BE, D2), lambda i: (i + off, 0)),
                pl.BlockSpec((BE, D2), lambda i: (i + off, 0)),
                pl.BlockSpec((BE, D2), lambda i: (i, 0)),
                pl.BlockSpec((D2, H), lambda i: (0, 0)),
                pl.BlockSpec((1, H), lambda i: (0, 0)),
                pl.BlockSpec((H, 2), lambda i: (0, 0)),
                pl.BlockSpec((1, 2), lambda i: (0, 0)),
            ],
            out_specs=pl.BlockSpec((BE, 2), lambda i: (i, 0)),
            out_shape=jax.ShapeDtypeStruct((N // 2, 2), f32),
        )(flat_pair_enc, key1, kbar_h, phi2_W0, b(phi2_b0), phi2_W1,
          b(phi2_b1))

    altq0 = _run_e(kbar0, 0)
    altq1 = _run_e(kbar1, 1)
    altq = jnp.concatenate([altq0, altq1], axis=0)
    return (qjt, altq)


# speedup vs baseline: 3.4779x; 1.0077x over previous
import functools

import jax
import jax.numpy as jnp
from jax import lax
from jax.experimental import pallas as pl
from jax.experimental.pallas import tpu as pltpu
from jax.experimental.pallas import tpu_sc as plsc

G = 10000
GA = 10624
W = 128
BA = 2560
RB = 128
BE = 6400


def _elu(x):
    return jnp.where(x > 0, x, jnp.exp(jnp.minimum(x, 0.0)) - 1.0)


def _phase_abc_body(enca_ref, gi_ref, w0_ref, b0_ref, w1_ref, b1_ref,
                    w2_ref, b2_ref, gw0_ref, gb0_ref, gw1_ref, gb1_ref,
                    gw2_ref, gb2_ref, key1_ref, qjt_ref, means_ref,
                    acc, cnt):
    f32 = jnp.float32
    pid = pl.program_id(0)
    nb = pl.num_programs(0)

    @pl.when(pid == 0)
    def _():
        acc[...] = jnp.zeros(acc.shape, f32)
        cnt[...] = jnp.zeros(cnt.shape, f32)

    x = enca_ref[...]
    h = _elu(jnp.dot(x, w0_ref[...], preferred_element_type=f32)
             + b0_ref[...])
    h = _elu(jnp.dot(h, w1_ref[...], preferred_element_type=f32)
             + b1_ref[...])
    k1 = jnp.dot(h, w2_ref[...], preferred_element_type=f32) + b2_ref[...]
    key1_ref[...] = k1

    gi_row = gi_ref[0]
    gmin = jnp.min(gi_row)
    gmax = jnp.max(gi_row)
    onesb = jnp.ones((BA, 8), f32)

    def window(b_):
        b8 = pl.multiple_of(b_, 8)
        ids = b8 + lax.broadcasted_iota(jnp.int32, (W, 1), 0)
        oh = (ids == gi_row).astype(f32)
        part = jnp.dot(oh, k1, preferred_element_type=f32)
        pc = jnp.dot(oh, onesb, preferred_element_type=f32)
        acc[pl.ds(b8, W), :] = acc[pl.ds(b8, W), :] + part
        cnt[pl.ds(b8, W), :] = cnt[pl.ds(b8, W), :] + pc
        return b_ + W

    base0 = (gmin // 8) * 8
    window(base0)

    def cond(b_):
        return b_ <= gmax

    lax.while_loop(cond, window, base0 + W)

    @pl.when(pid == nb - 1)
    def _():
        sums = acc[0:G, :]
        c = cnt[0:G, 0:1]
        means = sums / jnp.maximum(c, 1.0)
        q = _elu(jnp.dot(means, gw0_ref[...], preferred_element_type=f32)
                 + gb0_ref[...])
        q = _elu(jnp.dot(q, gw1_ref[...], preferred_element_type=f32)
                 + gb1_ref[...])
        qjt_ref[...] = (jnp.dot(q, gw2_ref[...], preferred_element_type=f32)
                        + gb2_ref[...])
        means_ref[...] = means


def _phase_e_body(enc_ref, key1_ref, kbar_ref, pw0_ref, pb0_ref, pw1_ref,
                  pb1_ref, out_ref):
    d = enc_ref[...] + kbar_ref[...] - key1_ref[...]
    pre = (jnp.dot(d, pw0_ref[...], preferred_element_type=jnp.float32) + pb0_ref[...])
    out_ref[...] = (jnp.dot(_elu(pre), pw1_ref[...],
                            preferred_element_type=jnp.float32) + pb1_ref[...])


def kernel(flat_pair_enc, flat_pair_encA, group_index,
           phi1_W0, phi1_b0, phi1_W1, phi1_b1, phi1_W2, phi1_b2,
           g_W0, g_b0, g_W1, g_b1, g_W2, g_b2,
           phi2_W0, phi2_b0, phi2_W1, phi2_b1):
    N, DA = flat_pair_encA.shape
    H = phi1_W1.shape[0]
    D2 = 2 * H
    f32 = jnp.float32
    i32 = jnp.int32

    info = plsc.get_sparse_core_info()
    NC, NS = info.num_cores, info.num_subcores
    NW = NC * NS
    NBLK = N // RB
    T = -(-NBLK // NW)
    NB = N // BA

    b = lambda v: v.reshape(1, -1)
    gi3 = group_index.reshape(NB, 1, BA)

    key1, qjt, means = pl.pallas_call(
        _phase_abc_body,
        grid=(NB,),
        in_specs=[
            pl.BlockSpec((BA, DA), lambda i: (i, 0)),
            pl.BlockSpec((1, 1, BA), lambda i: (i, 0, 0)),
            pl.BlockSpec((DA, H), lambda i: (0, 0)),
            pl.BlockSpec((1, H), lambda i: (0, 0)),
            pl.BlockSpec((H, H), lambda i: (0, 0)),
            pl.BlockSpec((1, H), lambda i: (0, 0)),
            pl.BlockSpec((H, D2), lambda i: (0, 0)),
            pl.BlockSpec((1, D2), lambda i: (0, 0)),
            pl.BlockSpec((D2, H), lambda i: (0, 0)),
            pl.BlockSpec((1, H), lambda i: (0, 0)),
            pl.BlockSpec((H, H), lambda i: (0, 0)),
            pl.BlockSpec((1, H), lambda i: (0, 0)),
            pl.BlockSpec((H, 1), lambda i: (0, 0)),
            pl.BlockSpec((1, 1), lambda i: (0, 0)),
        ],
        out_specs=[
            pl.BlockSpec((BA, D2), lambda i: (i, 0)),
            pl.BlockSpec((G, 1), lambda i: (0, 0)),
            pl.BlockSpec((G, D2), lambda i: (0, 0)),
        ],
        out_shape=[
            jax.ShapeDtypeStruct((N, D2), f32),
            jax.ShapeDtypeStruct((G, 1), f32),
            jax.ShapeDtypeStruct((G, D2), f32),
        ],
        scratch_shapes=[
            pltpu.VMEM((GA, D2), f32),
            pltpu.VMEM((GA, 8), f32),
        ],
    )(flat_pair_encA, gi3, phi1_W0, b(phi1_b0), phi1_W1, b(phi1_b1),
      phi1_W2, b(phi1_b2), g_W0, b(g_b0), g_W1, b(g_b1), g_W2,
      g_b2.reshape(1, 1))

    mesh = plsc.VectorSubcoreMesh(core_axis_name="c", subcore_axis_name="s")

    SB = 4 * RB
    NSBH = (N // 2) // SB
    TSB = -(-NSBH // NW)

    def _make_gather(base):
        @functools.partial(
            pl.kernel, mesh=mesh,
            out_type=jax.ShapeDtypeStruct((N // 2, D2), f32),
            scratch_types=[
                pltpu.VMEM((SB,), i32),
                pltpu.VMEM((SB, D2), f32),
                pltpu.SemaphoreType.DMA,
            ],
        )
        def _sc_gather(tab_hbm, gi_hbm, out_hbm, idx_v, rows_v, sem):
            cid = lax.axis_index("c")
            sid = lax.axis_index("s")
            wid = sid * NC + cid

            def trip(t, cc):
                sb = t * NW + wid

                @pl.when(sb < NSBH)
                def _():
                    r0 = sb * SB
                    pltpu.sync_copy(gi_hbm.at[pl.ds(base + r0, SB)], idx_v)
                    cps = [
                        pltpu.async_copy(
                            tab_hbm.at[idx_v.at[pl.ds(k * RB, RB)]],
                            rows_v.at[pl.ds(k * RB, RB)], sem)
                        for k in range(SB // RB)
                    ]
                    for cp in cps:
                        cp.wait()
                    pltpu.sync_copy(rows_v, out_hbm.at[pl.ds(r0, SB)])
                return cc

            lax.fori_loop(0, TSB, trip, 0)

        return _sc_gather

    kbar0 = _make_gather(0)(means, group_index)
    kbar1 = _make_gather(N // 2)(means, group_index)

    NEH = (N // 2) // BE

    def _run_e(kbar_h, half):
        off = half * NEH
        return pl.pallas_call(
            _phase_e_body,
            grid=(NEH,),
            in_specs=[
                pl.BlockSpec((BE, D2), lambda i: (i + off, 0)),
                pl.BlockSpec((BE, D2), lambda i: (i + off, 0)),
                pl.BlockSpec((BE, D2), lambda i: (i, 0)),
                pl.BlockSpec((D2, H), lambda i: (0, 0)),
                pl.BlockSpec((1, H), lambda i: (0, 0)),
                pl.BlockSpec((H, 2), lambda i: (0, 0)),
                pl.BlockSpec((1, 2), lambda i: (0, 0)),
            ],
            out_specs=pl.BlockSpec((BE, 2), lambda i: (i, 0)),
            out_shape=jax.ShapeDtypeStruct((N // 2, 2), f32),
        )(flat_pair_enc, key1, kbar_h, phi2_W0, b(phi2_b0), phi2_W1,
          b(phi2_b1))

    altq0 = _run_e(kbar0, 0)
    altq1 = _run_e(kbar1, 1)
    altq = jnp.concatenate([altq0, altq1], axis=0)
    return (qjt, altq)
